# baseline (speedup 1.0000x reference)
import jax
import jax.numpy as jnp
from jax import lax
from jax.experimental import pallas as pl
from jax.experimental.pallas import tpu as pltpu
from jax.experimental.pallas import tpu_sc as plsc

_CV = 0.1
_SV = 0.2
_CONF = 0.01
_NMS_T = 0.5
_MAXOUT = 100
_N = 20000
_NW = 16
_AW = _N // _NW
_CHUNK = 1280
_NP = _CHUNK * _NW
_VPC = _CHUNK // 16
_NBLK = _NP // 16
_NSUP = _NBLK // 16
_SELP = 112
_CLS_SLAB = 2504
_BOX_SLAB = 5000
_SMOFF = _NBLK
_OUTP = 560
_NEG = -1e9
_BIG = 1 << 30


def _iota16():
    return lax.broadcasted_iota(jnp.int32, (16,), 0)


def _lane(v, lane_idx, fill):
    return jnp.max(jnp.where(_iota16() == lane_idx, v, fill))


def _sc_body(clsh, boxh, anch, dimh,
             out_hbm,
             clsb, boxb, anchb, dimv, fillb,
             rs, rx0, ry0, rx1, ry1,
             fsc, fx0, fy0, fx1, fy1,
             sx0, sy0, sx1, sy1, ssc,
             sh0, sh1, sh2, sh3, sh4, shb, sem):
    w = lax.axis_index("s")
    it = _iota16()

    cstart = pl.multiple_of((w * 2 * _AW) // 8 * 8, 8)
    phase = w * 2 * _AW - cstart
    bstart = pl.multiple_of(w * 4 * _AW, 8)
    cops = (
        pltpu.async_copy(clsh.at[pl.ds(cstart, _CLS_SLAB)], clsb, sem),
        pltpu.async_copy(boxh.at[pl.ds(bstart, _BOX_SLAB)], boxb, sem),
        pltpu.async_copy(anch.at[pl.ds(bstart, _BOX_SLAB)], anchb, sem),
        pltpu.async_copy(dimh, dimv, sem),
    )
    for c in cops:
        c.wait()
    dv = dimv[...]
    sw = jnp.max(jnp.where(it == 0, dv, _NEG))
    sh = jnp.max(jnp.where(it == 1, dv, _NEG))

    def decode(i, carry):
        aidx = i * 16 + it
        valid = aidx < _AW
        ci = jnp.minimum(phase + 2 * aidx, _CLS_SLAB - 2)
        bi = jnp.minimum(4 * aidx, _BOX_SLAB - 4)
        a = plsc.load_gather(clsb, [ci])
        b = plsc.load_gather(clsb, [ci + 1])
        tcx = plsc.load_gather(boxb, [bi])
        tcy = plsc.load_gather(boxb, [bi + 1])
        tw = plsc.load_gather(boxb, [bi + 2])
        th = plsc.load_gather(boxb, [bi + 3])
        acx = plsc.load_gather(anchb, [bi])
        acy = plsc.load_gather(anchb, [bi + 1])
        aw = plsc.load_gather(anchb, [bi + 2])
        ah = plsc.load_gather(anchb, [bi + 3])
        mx = jnp.maximum(a, b)
        e0 = jnp.exp(a - mx)
        e1 = jnp.exp(b - mx)
        fg = e1 / (e0 + e1)
        cx = tcx * _CV * aw + acx
        cy = tcy * _CV * ah + acy
        bw = jnp.exp(tw * _SV) * aw
        bh = jnp.exp(th * _SV) * ah
        masked = jnp.where(valid & (fg > _CONF), fg, _NEG)
        off = pl.multiple_of(i * 16, 16)
        sl = pl.ds(off, 16)
        rs[sl] = masked
        rx0[sl] = (cx - bw / 2.0) * sw
        ry0[sl] = (cy - bh / 2.0) * sh
        rx1[sl] = (cx + bw / 2.0) * sw
        ry1[sl] = (cy + bh / 2.0) * sh
        return carry

    lax.fori_loop(0, _VPC, decode, 0)

    a0 = plsc.load_gather(clsb, [phase + 2 * it])
    a1 = plsc.load_gather(clsb, [phase + 2 * it + 1])
    mx0 = jnp.maximum(a0, a1)
    fillb[...] = jnp.exp(a1 - mx0) / (jnp.exp(a0 - mx0) + jnp.exp(a1 - mx0))

    for s in range(_VPC // 16):
        acc = plsc.load_gather(rs, [s * 256 + it * 16])
        for j in range(1, 16):
            acc = jnp.maximum(acc,
                              plsc.load_gather(rs, [s * 256 + it * 16 + j]))
        clsb[pl.ds(s * 16, 16)] = acc

    base = w * _CHUNK
    sops = (
        pltpu.async_copy(rs, sh0.at[pl.ds(base, _CHUNK)], sem),
        pltpu.async_copy(rx0, sh1.at[pl.ds(base, _CHUNK)], sem),
        pltpu.async_copy(ry0, sh2.at[pl.ds(base, _CHUNK)], sem),
        pltpu.async_copy(rx1, sh3.at[pl.ds(base, _CHUNK)], sem),
        pltpu.async_copy(ry1, sh4.at[pl.ds(base, _CHUNK)], sem),
        pltpu.async_copy(clsb.at[pl.ds(0, 80)],
                         shb.at[pl.ds(w * 80, 80)], sem),
    )
    for c in sops:
        c.wait()
    plsc.subcore_barrier()

    @pl.when(w == 0)
    def _phase_b():
        gops = (
            pltpu.async_copy(sh0, fsc, sem),
            pltpu.async_copy(sh1, fx0, sem),
            pltpu.async_copy(sh2, fy0, sem),
            pltpu.async_copy(sh3, fx1, sem),
            pltpu.async_copy(sh4, fy1, sem),
            pltpu.async_copy(shb, boxb.at[pl.ds(0, _NBLK)], sem),
        )
        for c in gops:
            c.wait()

        for si in range(_NSUP // 16):
            gbase = si * 256
            acc = plsc.load_gather(boxb, [gbase + it * 16])
            for j in range(1, 16):
                acc = jnp.maximum(
                    acc, plsc.load_gather(boxb, [gbase + it * 16 + j]))
            boxb[pl.ds(_SMOFF + si * 16, 16)] = acc

        c0 = _lane(fx0[pl.ds(0, 16)], 0, _NEG)
        c1 = _lane(fy0[pl.ds(0, 16)], 0, _NEG)
        c2 = _lane(fx1[pl.ds(0, 16)], 0, _NEG)
        c3 = _lane(fy1[pl.ds(0, 16)], 0, _NEG)
        c4 = _lane(fillb[...], 0, _NEG)
        for j in range(_SELP // 16):
            sl = pl.ds(j * 16, 16)
            sx0[sl] = jnp.broadcast_to(c0, (16,))
            sy0[sl] = jnp.broadcast_to(c1, (16,))
            sx1[sl] = jnp.broadcast_to(c2, (16,))
            sy1[sl] = jnp.broadcast_to(c3, (16,))
            ssc[sl] = jnp.broadcast_to(c4, (16,))

        def global_max():
            acc = boxb[pl.ds(_SMOFF, 16)]
            for si in range(1, _NSUP // 16):
                acc = jnp.maximum(acc, boxb[pl.ds(_SMOFF + si * 16, 16)])
            return jnp.max(acc)

        def cond(state):
            k, m = state
            return jnp.logical_and(k < _MAXOUT, m > _NEG)

        def body(state):
            k, m = state
            sacc = jnp.full((16,), _BIG, jnp.int32)
            for si in range(_NSUP // 16):
                v = boxb[pl.ds(_SMOFF + si * 16, 16)]
                sacc = jnp.minimum(sacc,
                                   jnp.where(v == m, si * 16 + it, _BIG))
            s = jnp.min(sacc)
            bv = boxb[pl.ds(pl.multiple_of(s * 16, 16), 16)]
            bnum = jnp.min(jnp.where(bv == m, s * 16 + it, _BIG))
            eoff = pl.multiple_of(bnum * 16, 16)
            ev = fsc[pl.ds(eoff, 16)]
            lane_g = jnp.min(jnp.where(ev == m, it, _BIG))

            hit = it == lane_g
            cx0 = jnp.max(jnp.where(hit, fx0[pl.ds(eoff, 16)], _NEG))
            cy0 = jnp.max(jnp.where(hit, fy0[pl.ds(eoff, 16)], _NEG))
            cx1 = jnp.max(jnp.where(hit, fx1[pl.ds(eoff, 16)], _NEG))
            cy1 = jnp.max(jnp.where(hit, fy1[pl.ds(eoff, 16)], _NEG))
            a1 = (jnp.maximum(cx1 - cx0, 0.0) *
                  jnp.maximum(cy1 - cy0, 0.0))

            def iou_step(j, acc):
                off = pl.multiple_of(j * 16, 16)
                tx0 = sx0[pl.ds(off, 16)]
                ty0 = sy0[pl.ds(off, 16)]
                tx1 = sx1[pl.ds(off, 16)]
                ty1 = sy1[pl.ds(off, 16)]
                iw = jnp.maximum(
                    jnp.minimum(cx1, tx1) - jnp.maximum(cx0, tx0), 0.0)
                ih = jnp.maximum(
                    jnp.minimum(cy1, ty1) - jnp.maximum(cy0, ty0), 0.0)
                inter = iw * ih
                a2 = (jnp.maximum(tx1 - tx0, 0.0) *
                      jnp.maximum(ty1 - ty0, 0.0))
                iou = inter / (a1 + a2 - inter + 1e-9)
                valid = (j * 16 + it) < k
                return jnp.maximum(acc, jnp.where(valid, iou, 0.0))

            nv = (k + 15) // 16
            iou_max = lax.fori_loop(0, nv, iou_step,
                                    jnp.zeros((16,), jnp.float32))
            supp = jnp.max(iou_max) > _NMS_T

            @pl.when(jnp.logical_not(supp))
            def _insert():
                koff = pl.multiple_of((k // 16) * 16, 16)
                klane = k & 15
                put = it == klane
                ksl = pl.ds(koff, 16)
                sx0[ksl] = jnp.where(put, cx0, sx0[ksl])
                sy0[ksl] = jnp.where(put, cy0, sy0[ksl])
                sx1[ksl] = jnp.where(put, cx1, sx1[ksl])
                sy1[ksl] = jnp.where(put, cy1, sy1[ksl])
                ssc[ksl] = jnp.where(put, m, ssc[ksl])

            k = k + jnp.where(supp, 0, 1).astype(jnp.int32)

            nev = jnp.where(hit, _NEG, ev)
            fsc[pl.ds(eoff, 16)] = nev
            nbv = jnp.where(it == (bnum & 15), jnp.max(nev), bv)
            boxb[pl.ds(pl.multiple_of(s * 16, 16), 16)] = nbv
            soff = pl.multiple_of(_SMOFF + (s // 16) * 16, 16)
            ssl = pl.ds(soff, 16)
            nsv = jnp.where(it == (s & 15), jnp.max(nbv), boxb[ssl])
            boxb[ssl] = nsv
            return k, global_max()

        lax.while_loop(cond, body, (jnp.int32(0), global_max()))

        for j in range(_SELP // 16):
            sl = pl.ds(j * 16, 16)
            rowbase = (j * 16 + it) * 5
            plsc.store_scatter(rs, [rowbase], sx0[sl])
            plsc.store_scatter(rs, [rowbase + 1], sy0[sl])
            plsc.store_scatter(rs, [rowbase + 2], sx1[sl])
            plsc.store_scatter(rs, [rowbase + 3], sy1[sl])
            plsc.store_scatter(rs, [rowbase + 4], ssc[sl])
        pltpu.sync_copy(rs.at[pl.ds(0, _OUTP)], out_hbm)


@jax.jit
def kernel(cls_logits, bbox_pred, anchors, image_h, image_w):
    clsf = jnp.pad(cls_logits.reshape(-1), (0, 32))
    boxf = jnp.pad(bbox_pred.reshape(-1), (0, 32))
    anchf = jnp.pad(anchors.reshape(-1), (0, 32))
    dims = jnp.zeros((16,), jnp.float32)
    dims = dims.at[0].set(jnp.float32(image_w)).at[1].set(jnp.float32(image_h))

    mesh = plsc.VectorSubcoreMesh(core_axis_name="c", subcore_axis_name="s",
                                  num_cores=1)
    run = pl.kernel(
        _sc_body,
        out_type=jax.ShapeDtypeStruct((_OUTP,), jnp.float32),
        mesh=mesh,
        compiler_params=pltpu.CompilerParams(needs_layout_passes=False),
        scratch_types=(
            [pltpu.VMEM((_CLS_SLAB,), jnp.float32),
             pltpu.VMEM((_BOX_SLAB,), jnp.float32),
             pltpu.VMEM((_BOX_SLAB,), jnp.float32),
             pltpu.VMEM((16,), jnp.float32), pltpu.VMEM((16,), jnp.float32)] +
            [pltpu.VMEM((_CHUNK,), jnp.float32) for _ in range(5)] +
            [pltpu.VMEM((_NP,), jnp.float32) for _ in range(5)] +
            [pltpu.VMEM((_SELP,), jnp.float32) for _ in range(5)] +
            [pltpu.VMEM_SHARED((_NP,), jnp.float32) for _ in range(5)] +
            [pltpu.VMEM_SHARED((_NBLK,), jnp.float32),
             pltpu.SemaphoreType.DMA]
        ),
    )
    out = run(clsf, boxf, anchf, dims)
    return out[:_MAXOUT * 5].reshape(_MAXOUT, 5)


# speedup vs baseline: 1.7151x; 1.7151x over previous
import jax
import jax.numpy as jnp
from jax import lax
from jax.experimental import pallas as pl
from jax.experimental.pallas import tpu as pltpu
from jax.experimental.pallas import tpu_sc as plsc

_CV = 0.1
_SV = 0.2
_CONF = 0.01
_NMS_T = 0.5
_MAXOUT = 100
_N = 20000
_NP = 20480
_NW = 16
_CHUNK = _NP // _NW
_VPC = _CHUNK // 16
_NBLK = _NP // 16
_NSUP = _NBLK // 16
_SELP = 112
_OUTP = 560
_NEG = -1e9
_BIG = 1 << 30


def _iota16():
    return lax.broadcasted_iota(jnp.int32, (16,), 0)


def _lane(v, lane_idx, fill):
    return jnp.max(jnp.where(_iota16() == lane_idx, v, fill))


def _sc_body(l0h, l1h, tch, tcyh, twh, thh, acxh, acyh, awh, ahh, dimh,
             out_hbm,
             b0, b1, b2, b3, b4, b5, b6, b7, b8, b9, dimv, fillb,
             fsc, fx0, fy0, fx1, fy1,
             sx0, sy0, sx1, sy1, ssc,
             sh0, sh1, sh2, sh3, sh4, shb, sem):
    w = lax.axis_index("s")
    base = w * _CHUNK
    it = _iota16()

    cops = tuple(
        pltpu.async_copy(hbm.at[pl.ds(base, _CHUNK)], ref, sem)
        for ref, hbm in ((b0, l0h), (b1, l1h), (b2, tch), (b3, tcyh),
                         (b4, twh), (b5, thh), (b6, acxh), (b7, acyh),
                         (b8, awh), (b9, ahh))
    ) + (pltpu.async_copy(dimh, dimv, sem),)
    for c in cops:
        c.wait()
    dv = dimv[...]
    sw = jnp.max(jnp.where(it == 0, dv, _NEG))
    sh = jnp.max(jnp.where(it == 1, dv, _NEG))

    a0 = b0[pl.ds(0, 16)]
    a1 = b1[pl.ds(0, 16)]
    mx0 = jnp.maximum(a0, a1)
    fillb[...] = jnp.exp(a1 - mx0) / (jnp.exp(a0 - mx0) + jnp.exp(a1 - mx0))

    def decode(i, carry):
        off = pl.multiple_of(i * 16, 16)
        sl = pl.ds(off, 16)
        a = b0[sl]
        b = b1[sl]
        tcx = b2[sl]
        tcy = b3[sl]
        tw = b4[sl]
        th = b5[sl]
        acx = b6[sl]
        acy = b7[sl]
        aw = b8[sl]
        ah = b9[sl]
        mx = jnp.maximum(a, b)
        e0 = jnp.exp(a - mx)
        e1 = jnp.exp(b - mx)
        fg = e1 / (e0 + e1)
        cx = tcx * _CV * aw + acx
        cy = tcy * _CV * ah + acy
        bw = jnp.exp(tw * _SV) * aw
        bh = jnp.exp(th * _SV) * ah
        gidx = base + off + it
        masked = jnp.where((fg > _CONF) & (gidx < _N), fg, _NEG)
        b0[sl] = masked
        b1[sl] = (cx - bw / 2.0) * sw
        b2[sl] = (cy - bh / 2.0) * sh
        b3[sl] = (cx + bw / 2.0) * sw
        b4[sl] = (cy + bh / 2.0) * sh
        return carry

    lax.fori_loop(0, _VPC, decode, 0)

    for s in range(_VPC // 16):
        acc = plsc.load_gather(b0, [s * 256 + it * 16])
        for j in range(1, 16):
            acc = jnp.maximum(acc,
                              plsc.load_gather(b0, [s * 256 + it * 16 + j]))
        b5[pl.ds(s * 16, 16)] = acc

    sops = (
        pltpu.async_copy(b0, sh0.at[pl.ds(base, _CHUNK)], sem),
        pltpu.async_copy(b1, sh1.at[pl.ds(base, _CHUNK)], sem),
        pltpu.async_copy(b2, sh2.at[pl.ds(base, _CHUNK)], sem),
        pltpu.async_copy(b3, sh3.at[pl.ds(base, _CHUNK)], sem),
        pltpu.async_copy(b4, sh4.at[pl.ds(base, _CHUNK)], sem),
        pltpu.async_copy(b5.at[pl.ds(0, _VPC)],
                         shb.at[pl.ds(w * _VPC, _VPC)], sem),
    )
    for c in sops:
        c.wait()
    plsc.subcore_barrier()

    @pl.when(w == 0)
    def _phase_b():
        gops = (
            pltpu.async_copy(sh0, fsc, sem),
            pltpu.async_copy(sh1, fx0, sem),
            pltpu.async_copy(sh2, fy0, sem),
            pltpu.async_copy(sh3, fx1, sem),
            pltpu.async_copy(sh4, fy1, sem),
            pltpu.async_copy(shb, b0, sem),
        )
        for c in gops:
            c.wait()

        for si in range(_NSUP // 16):
            gbase = si * 256
            acc = plsc.load_gather(b0, [gbase + it * 16])
            for j in range(1, 16):
                acc = jnp.maximum(
                    acc, plsc.load_gather(b0, [gbase + it * 16 + j]))
            b1[pl.ds(si * 16, 16)] = acc

        c0 = _lane(fx0[pl.ds(0, 16)], 0, _NEG)
        c1 = _lane(fy0[pl.ds(0, 16)], 0, _NEG)
        c2 = _lane(fx1[pl.ds(0, 16)], 0, _NEG)
        c3 = _lane(fy1[pl.ds(0, 16)], 0, _NEG)
        c4 = _lane(fillb[...], 0, _NEG)
        for j in range(_SELP // 16):
            sl = pl.ds(j * 16, 16)
            sx0[sl] = jnp.broadcast_to(c0, (16,))
            sy0[sl] = jnp.broadcast_to(c1, (16,))
            sx1[sl] = jnp.broadcast_to(c2, (16,))
            sy1[sl] = jnp.broadcast_to(c3, (16,))
            ssc[sl] = jnp.broadcast_to(c4, (16,))

        def global_max():
            acc = b1[pl.ds(0, 16)]
            for si in range(1, _NSUP // 16):
                acc = jnp.maximum(acc, b1[pl.ds(si * 16, 16)])
            return jnp.max(acc)

        def cond(state):
            k, m = state
            return jnp.logical_and(k < _MAXOUT, m > _NEG)

        def body(state):
            k, m = state
            sacc = jnp.full((16,), _BIG, jnp.int32)
            for si in range(_NSUP // 16):
                v = b1[pl.ds(si * 16, 16)]
                sacc = jnp.minimum(sacc,
                                   jnp.where(v == m, si * 16 + it, _BIG))
            s = jnp.min(sacc)
            bv = b0[pl.ds(pl.multiple_of(s * 16, 16), 16)]
            bnum = jnp.min(jnp.where(bv == m, s * 16 + it, _BIG))
            eoff = pl.multiple_of(bnum * 16, 16)
            ev = fsc[pl.ds(eoff, 16)]
            lane_g = jnp.min(jnp.where(ev == m, it, _BIG))

            hit = it == lane_g
            cx0 = jnp.max(jnp.where(hit, fx0[pl.ds(eoff, 16)], _NEG))
            cy0 = jnp.max(jnp.where(hit, fy0[pl.ds(eoff, 16)], _NEG))
            cx1 = jnp.max(jnp.where(hit, fx1[pl.ds(eoff, 16)], _NEG))
            cy1 = jnp.max(jnp.where(hit, fy1[pl.ds(eoff, 16)], _NEG))
            a1 = (jnp.maximum(cx1 - cx0, 0.0) *
                  jnp.maximum(cy1 - cy0, 0.0))

            def iou_step(j, acc):
                off = pl.multiple_of(j * 16, 16)
                tx0 = sx0[pl.ds(off, 16)]
                ty0 = sy0[pl.ds(off, 16)]
                tx1 = sx1[pl.ds(off, 16)]
                ty1 = sy1[pl.ds(off, 16)]
                iw = jnp.maximum(
                    jnp.minimum(cx1, tx1) - jnp.maximum(cx0, tx0), 0.0)
                ih = jnp.maximum(
                    jnp.minimum(cy1, ty1) - jnp.maximum(cy0, ty0), 0.0)
                inter = iw * ih
                a2 = (jnp.maximum(tx1 - tx0, 0.0) *
                      jnp.maximum(ty1 - ty0, 0.0))
                iou = inter / (a1 + a2 - inter + 1e-9)
                valid = (j * 16 + it) < k
                return jnp.maximum(acc, jnp.where(valid, iou, 0.0))

            nv = (k + 15) // 16
            iou_max = lax.fori_loop(0, nv, iou_step,
                                    jnp.zeros((16,), jnp.float32))
            supp = jnp.max(iou_max) > _NMS_T

            @pl.when(jnp.logical_not(supp))
            def _insert():
                koff = pl.multiple_of((k // 16) * 16, 16)
                klane = k & 15
                put = it == klane
                ksl = pl.ds(koff, 16)
                sx0[ksl] = jnp.where(put, cx0, sx0[ksl])
                sy0[ksl] = jnp.where(put, cy0, sy0[ksl])
                sx1[ksl] = jnp.where(put, cx1, sx1[ksl])
                sy1[ksl] = jnp.where(put, cy1, sy1[ksl])
                ssc[ksl] = jnp.where(put, m, ssc[ksl])

            k = k + jnp.where(supp, 0, 1).astype(jnp.int32)

            nev = jnp.where(hit, _NEG, ev)
            fsc[pl.ds(eoff, 16)] = nev
            nbv = jnp.where(it == (bnum & 15), jnp.max(nev), bv)
            b0[pl.ds(pl.multiple_of(s * 16, 16), 16)] = nbv
            soff = pl.multiple_of((s // 16) * 16, 16)
            ssl = pl.ds(soff, 16)
            nsv = jnp.where(it == (s & 15), jnp.max(nbv), b1[ssl])
            b1[ssl] = nsv
            return k, global_max()

        lax.while_loop(cond, body, (jnp.int32(0), global_max()))

        for j in range(_SELP // 16):
            sl = pl.ds(j * 16, 16)
            rowbase = (j * 16 + it) * 5
            plsc.store_scatter(b2, [rowbase], sx0[sl])
            plsc.store_scatter(b2, [rowbase + 1], sy0[sl])
            plsc.store_scatter(b2, [rowbase + 2], sx1[sl])
            plsc.store_scatter(b2, [rowbase + 3], sy1[sl])
            plsc.store_scatter(b2, [rowbase + 4], ssc[sl])
        pltpu.sync_copy(b2.at[pl.ds(0, _OUTP)], out_hbm)


def _plane(x):
    return jnp.pad(x, (0, _NP - _N))


@jax.jit
def kernel(cls_logits, bbox_pred, anchors, image_h, image_w):
    planes = [
        _plane(cls_logits[0, :, 0]), _plane(cls_logits[0, :, 1]),
        _plane(bbox_pred[0, :, 0]), _plane(bbox_pred[0, :, 1]),
        _plane(bbox_pred[0, :, 2]), _plane(bbox_pred[0, :, 3]),
        _plane(anchors[:, 0]), _plane(anchors[:, 1]),
        _plane(anchors[:, 2]), _plane(anchors[:, 3]),
    ]
    dims = jnp.zeros((16,), jnp.float32)
    dims = dims.at[0].set(jnp.float32(image_w)).at[1].set(jnp.float32(image_h))

    mesh = plsc.VectorSubcoreMesh(core_axis_name="c", subcore_axis_name="s",
                                  num_cores=1)
    run = pl.kernel(
        _sc_body,
        out_type=jax.ShapeDtypeStruct((_OUTP,), jnp.float32),
        mesh=mesh,
        compiler_params=pltpu.CompilerParams(needs_layout_passes=False),
        scratch_types=(
            [pltpu.VMEM((_CHUNK,), jnp.float32) for _ in range(10)] +
            [pltpu.VMEM((16,), jnp.float32), pltpu.VMEM((16,), jnp.float32)] +
            [pltpu.VMEM((_NP,), jnp.float32) for _ in range(5)] +
            [pltpu.VMEM((_SELP,), jnp.float32) for _ in range(5)] +
            [pltpu.VMEM_SHARED((_NP,), jnp.float32) for _ in range(5)] +
            [pltpu.VMEM_SHARED((_NBLK,), jnp.float32),
             pltpu.SemaphoreType.DMA]
        ),
    )
    out = run(*planes, dims)
    return out[:_MAXOUT * 5].reshape(_MAXOUT, 5)


# speedup vs baseline: 1.7848x; 1.0406x over previous
import jax
import jax.numpy as jnp
from jax import lax
from jax.experimental import pallas as pl
from jax.experimental.pallas import tpu as pltpu
from jax.experimental.pallas import tpu_sc as plsc

_CV = 0.1
_SV = 0.2
_CONF = 0.01
_NMS_T = 0.5
_MAXOUT = 100
_N = 20000
_NP = 20480
_NW = 16
_CHUNK = _NP // _NW
_VPC = _CHUNK // 16
_NBLK = _NP // 16
_NSUP = _NBLK // 16
_SELP = 112
_OUTP = 560
_NEG = -1e9
_BIG = 1 << 30


def _iota16():
    return lax.broadcasted_iota(jnp.int32, (16,), 0)


def _lane(v, lane_idx, fill):
    return jnp.max(jnp.where(_iota16() == lane_idx, v, fill))


def _sc_body(l0h, l1h, tch, tcyh, twh, thh, acxh, acyh, awh, ahh, dimh,
             out_hbm,
             b0, b1, b2, b3, b4, b5, b6, b7, b8, b9, dimv, fillb,
             fsc, fx0, fy0, fx1, fy1,
             sx0, sy0, sx1, sy1, ssc,
             sh0, sh1, sh2, sh3, sh4, shb, sem, semb):
    w = lax.axis_index("s")
    base = w * _CHUNK
    it = _iota16()

    cops = tuple(
        pltpu.async_copy(hbm.at[pl.ds(base, _CHUNK)], ref, sem)
        for ref, hbm in ((b0, l0h), (b1, l1h), (b2, tch), (b3, tcyh),
                         (b4, twh), (b5, thh), (b6, acxh), (b7, acyh),
                         (b8, awh), (b9, ahh))
    ) + (pltpu.async_copy(dimh, dimv, sem),)
    for c in cops:
        c.wait()
    dv = dimv[...]
    sw = jnp.max(jnp.where(it == 0, dv, _NEG))
    sh = jnp.max(jnp.where(it == 1, dv, _NEG))

    a0 = b0[pl.ds(0, 16)]
    a1 = b1[pl.ds(0, 16)]
    mx0 = jnp.maximum(a0, a1)
    fillb[...] = jnp.exp(a1 - mx0) / (jnp.exp(a0 - mx0) + jnp.exp(a1 - mx0))

    def decode(i, carry):
        off = pl.multiple_of(i * 16, 16)
        sl = pl.ds(off, 16)
        a = b0[sl]
        b = b1[sl]
        tcx = b2[sl]
        tcy = b3[sl]
        tw = b4[sl]
        th = b5[sl]
        acx = b6[sl]
        acy = b7[sl]
        aw = b8[sl]
        ah = b9[sl]
        mx = jnp.maximum(a, b)
        e0 = jnp.exp(a - mx)
        e1 = jnp.exp(b - mx)
        fg = e1 / (e0 + e1)
        cx = tcx * _CV * aw + acx
        cy = tcy * _CV * ah + acy
        bw = jnp.exp(tw * _SV) * aw
        bh = jnp.exp(th * _SV) * ah
        gidx = base + off + it
        masked = jnp.where((fg > _CONF) & (gidx < _N), fg, _NEG)
        b0[sl] = masked
        b1[sl] = (cx - bw / 2.0) * sw
        b2[sl] = (cy - bh / 2.0) * sh
        b3[sl] = (cx + bw / 2.0) * sw
        b4[sl] = (cy + bh / 2.0) * sh
        return carry

    lax.fori_loop(0, _VPC, decode, 0)

    for s in range(_VPC // 16):
        acc = plsc.load_gather(b0, [s * 256 + it * 16])
        for j in range(1, 16):
            acc = jnp.maximum(acc,
                              plsc.load_gather(b0, [s * 256 + it * 16 + j]))
        b5[pl.ds(s * 16, 16)] = acc

    sops = (
        pltpu.async_copy(b0, sh0.at[pl.ds(base, _CHUNK)], sem),
        pltpu.async_copy(b1, sh1.at[pl.ds(base, _CHUNK)], sem),
        pltpu.async_copy(b2, sh2.at[pl.ds(base, _CHUNK)], sem),
        pltpu.async_copy(b3, sh3.at[pl.ds(base, _CHUNK)], sem),
        pltpu.async_copy(b4, sh4.at[pl.ds(base, _CHUNK)], sem),
        pltpu.async_copy(b5.at[pl.ds(0, _VPC)],
                         shb.at[pl.ds(w * _VPC, _VPC)], sem),
    )
    for c in sops:
        c.wait()
    plsc.subcore_barrier()

    @pl.when(w == 0)
    def _phase_b():
        bop = pltpu.async_copy(shb, b0, semb)
        gops = (
            pltpu.async_copy(sh0, fsc, sem),
            pltpu.async_copy(sh1, fx0, sem),
            pltpu.async_copy(sh2, fy0, sem),
            pltpu.async_copy(sh3, fx1, sem),
            pltpu.async_copy(sh4, fy1, sem),
        )
        bop.wait()

        for si in range(_NSUP // 16):
            gbase = si * 256
            acc = plsc.load_gather(b0, [gbase + it * 16])
            for j in range(1, 16):
                acc = jnp.maximum(
                    acc, plsc.load_gather(b0, [gbase + it * 16 + j]))
            b1[pl.ds(si * 16, 16)] = acc

        r0 = jnp.max(b1[pl.ds(0, 16)])
        r1 = jnp.max(b1[pl.ds(16, 16)])
        r2 = jnp.max(b1[pl.ds(32, 16)])
        r3 = jnp.max(b1[pl.ds(48, 16)])
        r4 = jnp.max(b1[pl.ds(64, 16)])

        for c in gops:
            c.wait()

        empty = jnp.broadcast_to(jnp.float32(-1e6), (16,))
        for j in range(_SELP // 16):
            sl = pl.ds(j * 16, 16)
            sx0[sl] = empty
            sy0[sl] = empty
            sx1[sl] = empty
            sy1[sl] = empty
            ssc[sl] = empty

        def cond(state):
            k, m, _, _, _, _, _ = state
            return jnp.logical_and(k < _MAXOUT, m > _NEG)

        def body(state):
            k, m, r0, r1, r2, r3, r4 = state
            sreg = jnp.where(
                r0 == m, 0, jnp.where(
                    r1 == m, 16, jnp.where(
                        r2 == m, 32, jnp.where(r3 == m, 48, 64))))
            soff = pl.multiple_of(sreg, 16)
            v = b1[pl.ds(soff, 16)]
            s = jnp.min(jnp.where(v == m, sreg + it, _BIG))
            bv = b0[pl.ds(pl.multiple_of(s * 16, 16), 16)]
            bnum = jnp.min(jnp.where(bv == m, s * 16 + it, _BIG))
            eoff = pl.multiple_of(bnum * 16, 16)
            ev = fsc[pl.ds(eoff, 16)]
            lane_g = jnp.min(jnp.where(ev == m, it, _BIG))

            hit = it == lane_g
            cx0 = jnp.max(jnp.where(hit, fx0[pl.ds(eoff, 16)], _NEG))
            cy0 = jnp.max(jnp.where(hit, fy0[pl.ds(eoff, 16)], _NEG))
            cx1 = jnp.max(jnp.where(hit, fx1[pl.ds(eoff, 16)], _NEG))
            cy1 = jnp.max(jnp.where(hit, fy1[pl.ds(eoff, 16)], _NEG))
            a1 = (jnp.maximum(cx1 - cx0, 0.0) *
                  jnp.maximum(cy1 - cy0, 0.0))

            def iou_step(j, acc):
                off = pl.multiple_of(j * 16, 16)
                tx0 = sx0[pl.ds(off, 16)]
                ty0 = sy0[pl.ds(off, 16)]
                tx1 = sx1[pl.ds(off, 16)]
                ty1 = sy1[pl.ds(off, 16)]
                iw = jnp.maximum(
                    jnp.minimum(cx1, tx1) - jnp.maximum(cx0, tx0), 0.0)
                ih = jnp.maximum(
                    jnp.minimum(cy1, ty1) - jnp.maximum(cy0, ty0), 0.0)
                inter = iw * ih
                a2 = (jnp.maximum(tx1 - tx0, 0.0) *
                      jnp.maximum(ty1 - ty0, 0.0))
                iou = inter / (a1 + a2 - inter + 1e-9)
                return jnp.maximum(acc, iou)

            nv = (k + 15) // 16
            iou_max = lax.fori_loop(0, nv, iou_step,
                                    jnp.zeros((16,), jnp.float32))
            supp = jnp.max(iou_max) > _NMS_T

            @pl.when(jnp.logical_not(supp))
            def _insert():
                koff = pl.multiple_of((k // 16) * 16, 16)
                klane = k & 15
                put = it == klane
                ksl = pl.ds(koff, 16)
                sx0[ksl] = jnp.where(put, cx0, sx0[ksl])
                sy0[ksl] = jnp.where(put, cy0, sy0[ksl])
                sx1[ksl] = jnp.where(put, cx1, sx1[ksl])
                sy1[ksl] = jnp.where(put, cy1, sy1[ksl])
                ssc[ksl] = jnp.where(put, m, ssc[ksl])

            k = k + jnp.where(supp, 0, 1).astype(jnp.int32)

            nev = jnp.where(hit, _NEG, ev)
            fsc[pl.ds(eoff, 16)] = nev
            nbv = jnp.where(it == (bnum & 15), jnp.max(nev), bv)
            b0[pl.ds(pl.multiple_of(s * 16, 16), 16)] = nbv
            nsv = jnp.where(it == (s - sreg), jnp.max(nbv), v)
            b1[pl.ds(soff, 16)] = nsv
            rn = jnp.max(nsv)
            r0 = jnp.where(sreg == 0, rn, r0)
            r1 = jnp.where(sreg == 16, rn, r1)
            r2 = jnp.where(sreg == 32, rn, r2)
            r3 = jnp.where(sreg == 48, rn, r3)
            r4 = jnp.where(sreg == 64, rn, r4)
            m = jnp.maximum(jnp.maximum(jnp.maximum(r0, r1),
                                        jnp.maximum(r2, r3)), r4)
            return k, m, r0, r1, r2, r3, r4

        m0 = jnp.maximum(jnp.maximum(jnp.maximum(r0, r1),
                                     jnp.maximum(r2, r3)), r4)
        fin = lax.while_loop(cond, body,
                             (jnp.int32(0), m0, r0, r1, r2, r3, r4))
        kf = fin[0]

        c0 = _lane(fx0[pl.ds(0, 16)], 0, _NEG)
        c1 = _lane(fy0[pl.ds(0, 16)], 0, _NEG)
        c2 = _lane(fx1[pl.ds(0, 16)], 0, _NEG)
        c3 = _lane(fy1[pl.ds(0, 16)], 0, _NEG)
        c4 = _lane(fillb[...], 0, _NEG)

        for j in range(_SELP // 16):
            sl = pl.ds(j * 16, 16)
            live = (j * 16 + it) < kf
            rowbase = (j * 16 + it) * 5
            plsc.store_scatter(b2, [rowbase], jnp.where(live, sx0[sl], c0))
            plsc.store_scatter(b2, [rowbase + 1], jnp.where(live, sy0[sl], c1))
            plsc.store_scatter(b2, [rowbase + 2], jnp.where(live, sx1[sl], c2))
            plsc.store_scatter(b2, [rowbase + 3], jnp.where(live, sy1[sl], c3))
            plsc.store_scatter(b2, [rowbase + 4], jnp.where(live, ssc[sl], c4))
        pltpu.sync_copy(b2.at[pl.ds(0, _OUTP)], out_hbm)


def _plane(x):
    return jnp.pad(x, (0, _NP - _N))


@jax.jit
def kernel(cls_logits, bbox_pred, anchors, image_h, image_w):
    planes = [
        _plane(cls_logits[0, :, 0]), _plane(cls_logits[0, :, 1]),
        _plane(bbox_pred[0, :, 0]), _plane(bbox_pred[0, :, 1]),
        _plane(bbox_pred[0, :, 2]), _plane(bbox_pred[0, :, 3]),
        _plane(anchors[:, 0]), _plane(anchors[:, 1]),
        _plane(anchors[:, 2]), _plane(anchors[:, 3]),
    ]
    dims = jnp.zeros((16,), jnp.float32)
    dims = dims.at[0].set(jnp.float32(image_w)).at[1].set(jnp.float32(image_h))

    mesh = plsc.VectorSubcoreMesh(core_axis_name="c", subcore_axis_name="s",
                                  num_cores=1)
    run = pl.kernel(
        _sc_body,
        out_type=jax.ShapeDtypeStruct((_OUTP,), jnp.float32),
        mesh=mesh,
        compiler_params=pltpu.CompilerParams(needs_layout_passes=False),
        scratch_types=(
            [pltpu.VMEM((_CHUNK,), jnp.float32) for _ in range(10)] +
            [pltpu.VMEM((16,), jnp.float32), pltpu.VMEM((16,), jnp.float32)] +
            [pltpu.VMEM((_NP,), jnp.float32) for _ in range(5)] +
            [pltpu.VMEM((_SELP,), jnp.float32) for _ in range(5)] +
            [pltpu.VMEM_SHARED((_NP,), jnp.float32) for _ in range(5)] +
            [pltpu.VMEM_SHARED((_NBLK,), jnp.float32),
             pltpu.SemaphoreType.DMA, pltpu.SemaphoreType.DMA]
        ),
    )
    out = run(*planes, dims)
    return out[:_MAXOUT * 5].reshape(_MAXOUT, 5)


# speedup vs baseline: 2.0746x; 1.1624x over previous
import jax
import jax.numpy as jnp
from jax import lax
from jax.experimental import pallas as pl
from jax.experimental.pallas import tpu as pltpu
from jax.experimental.pallas import tpu_sc as plsc

_CV = 0.1
_SV = 0.2
_CONF = 0.01
_NMS_T = 0.5
_MAXOUT = 100
_N = 20000
_NP = 20480
_NW = 16
_CHUNK = _NP // _NW
_VPC = _CHUNK // 16
_NBLK = _NP // 16
_NSUP = _NBLK // 16
_SELP = 112
_OUTP = 560
_NEG = -1e9
_BIG = 1 << 30


def _iota16():
    return lax.broadcasted_iota(jnp.int32, (16,), 0)


def _lane(v, lane_idx, fill):
    return jnp.max(jnp.where(_iota16() == lane_idx, v, fill))


def _sc_body(ph, dimh,
             out_hbm,
             b0, b1, b2, b3, b4, b5, b6, b7, b8, b9, dimv, fillb,
             fsc, fx0, fy0, fx1, fy1,
             sx0, sy0, sx1, sy1, ssc,
             sh0, sh1, sh2, sh3, sh4, shb, sem, semb):
    w = lax.axis_index("s")
    base = w * _CHUNK
    it = _iota16()

    cops = tuple(
        pltpu.async_copy(ph.at[pl.ds(f * _NP + base, _CHUNK)], ref, sem)
        for f, ref in enumerate((b0, b1, b2, b3, b4, b5, b6, b7, b8, b9))
    ) + (pltpu.async_copy(dimh, dimv, sem),)
    for c in cops:
        c.wait()
    dv = dimv[...]
    sw = jnp.max(jnp.where(it == 0, dv, _NEG))
    sh = jnp.max(jnp.where(it == 1, dv, _NEG))

    a0 = b0[pl.ds(0, 16)]
    a1 = b1[pl.ds(0, 16)]
    mx0 = jnp.maximum(a0, a1)
    fillb[...] = jnp.exp(a1 - mx0) / (jnp.exp(a0 - mx0) + jnp.exp(a1 - mx0))

    def decode(i, carry):
        off = pl.multiple_of(i * 16, 16)
        sl = pl.ds(off, 16)
        a = b0[sl]
        b = b1[sl]
        tcx = b2[sl]
        tcy = b3[sl]
        tw = b4[sl]
        th = b5[sl]
        acx = b6[sl]
        acy = b7[sl]
        aw = b8[sl]
        ah = b9[sl]
        mx = jnp.maximum(a, b)
        e0 = jnp.exp(a - mx)
        e1 = jnp.exp(b - mx)
        fg = e1 / (e0 + e1)
        cx = tcx * _CV * aw + acx
        cy = tcy * _CV * ah + acy
        bw = jnp.exp(tw * _SV) * aw
        bh = jnp.exp(th * _SV) * ah
        gidx = base + off + it
        masked = jnp.where((fg > _CONF) & (gidx < _N), fg, _NEG)
        b0[sl] = masked
        b1[sl] = (cx - bw / 2.0) * sw
        b2[sl] = (cy - bh / 2.0) * sh
        b3[sl] = (cx + bw / 2.0) * sw
        b4[sl] = (cy + bh / 2.0) * sh
        return carry

    lax.fori_loop(0, _VPC, decode, 0)

    for s in range(_VPC // 16):
        acc = plsc.load_gather(b0, [s * 256 + it * 16])
        for j in range(1, 16):
            acc = jnp.maximum(acc,
                              plsc.load_gather(b0, [s * 256 + it * 16 + j]))
        b5[pl.ds(s * 16, 16)] = acc

    sops = (
        pltpu.async_copy(b0, sh0.at[pl.ds(base, _CHUNK)], sem),
        pltpu.async_copy(b1, sh1.at[pl.ds(base, _CHUNK)], sem),
        pltpu.async_copy(b2, sh2.at[pl.ds(base, _CHUNK)], sem),
        pltpu.async_copy(b3, sh3.at[pl.ds(base, _CHUNK)], sem),
        pltpu.async_copy(b4, sh4.at[pl.ds(base, _CHUNK)], sem),
        pltpu.async_copy(b5.at[pl.ds(0, _VPC)],
                         shb.at[pl.ds(w * _VPC, _VPC)], sem),
    )
    for c in sops:
        c.wait()
    plsc.subcore_barrier()

    @pl.when(w == 0)
    def _phase_b():
        bop = pltpu.async_copy(shb, b0, semb)
        gops = (
            pltpu.async_copy(sh0, fsc, sem),
            pltpu.async_copy(sh1, fx0, sem),
            pltpu.async_copy(sh2, fy0, sem),
            pltpu.async_copy(sh3, fx1, sem),
            pltpu.async_copy(sh4, fy1, sem),
        )
        bop.wait()

        for si in range(_NSUP // 16):
            gbase = si * 256
            acc = plsc.load_gather(b0, [gbase + it * 16])
            for j in range(1, 16):
                acc = jnp.maximum(
                    acc, plsc.load_gather(b0, [gbase + it * 16 + j]))
            b1[pl.ds(si * 16, 16)] = acc

        r0 = jnp.max(b1[pl.ds(0, 16)])
        r1 = jnp.max(b1[pl.ds(16, 16)])
        r2 = jnp.max(b1[pl.ds(32, 16)])
        r3 = jnp.max(b1[pl.ds(48, 16)])
        r4 = jnp.max(b1[pl.ds(64, 16)])

        for c in gops:
            c.wait()

        empty = jnp.broadcast_to(jnp.float32(-1e6), (16,))
        for j in range(_SELP // 16):
            sl = pl.ds(j * 16, 16)
            sx0[sl] = empty
            sy0[sl] = empty
            sx1[sl] = empty
            sy1[sl] = empty
            ssc[sl] = empty

        def cond(state):
            k, m, _, _, _, _, _ = state
            return jnp.logical_and(k < _MAXOUT, m > _NEG)

        def body(state):
            k, m, r0, r1, r2, r3, r4 = state
            sreg = jnp.where(
                r0 == m, 0, jnp.where(
                    r1 == m, 16, jnp.where(
                        r2 == m, 32, jnp.where(r3 == m, 48, 64))))
            soff = pl.multiple_of(sreg, 16)
            v = b1[pl.ds(soff, 16)]
            s = jnp.min(jnp.where(v == m, sreg + it, _BIG))
            bv = b0[pl.ds(pl.multiple_of(s * 16, 16), 16)]
            bnum = jnp.min(jnp.where(bv == m, s * 16 + it, _BIG))
            eoff = pl.multiple_of(bnum * 16, 16)
            ev = fsc[pl.ds(eoff, 16)]
            lane_g = jnp.min(jnp.where(ev == m, it, _BIG))

            hit = it == lane_g
            cx0 = jnp.max(jnp.where(hit, fx0[pl.ds(eoff, 16)], _NEG))
            cy0 = jnp.max(jnp.where(hit, fy0[pl.ds(eoff, 16)], _NEG))
            cx1 = jnp.max(jnp.where(hit, fx1[pl.ds(eoff, 16)], _NEG))
            cy1 = jnp.max(jnp.where(hit, fy1[pl.ds(eoff, 16)], _NEG))
            a1 = (jnp.maximum(cx1 - cx0, 0.0) *
                  jnp.maximum(cy1 - cy0, 0.0))

            def iou_step(j, acc):
                off = pl.multiple_of(j * 16, 16)
                tx0 = sx0[pl.ds(off, 16)]
                ty0 = sy0[pl.ds(off, 16)]
                tx1 = sx1[pl.ds(off, 16)]
                ty1 = sy1[pl.ds(off, 16)]
                iw = jnp.maximum(
                    jnp.minimum(cx1, tx1) - jnp.maximum(cx0, tx0), 0.0)
                ih = jnp.maximum(
                    jnp.minimum(cy1, ty1) - jnp.maximum(cy0, ty0), 0.0)
                inter = iw * ih
                a2 = (jnp.maximum(tx1 - tx0, 0.0) *
                      jnp.maximum(ty1 - ty0, 0.0))
                iou = inter / (a1 + a2 - inter + 1e-9)
                return jnp.maximum(acc, iou)

            nv = (k + 15) // 16
            iou_max = lax.fori_loop(0, nv, iou_step,
                                    jnp.zeros((16,), jnp.float32))
            supp = jnp.max(iou_max) > _NMS_T

            @pl.when(jnp.logical_not(supp))
            def _insert():
                koff = pl.multiple_of((k // 16) * 16, 16)
                klane = k & 15
                put = it == klane
                ksl = pl.ds(koff, 16)
                sx0[ksl] = jnp.where(put, cx0, sx0[ksl])
                sy0[ksl] = jnp.where(put, cy0, sy0[ksl])
                sx1[ksl] = jnp.where(put, cx1, sx1[ksl])
                sy1[ksl] = jnp.where(put, cy1, sy1[ksl])
                ssc[ksl] = jnp.where(put, m, ssc[ksl])

            k = k + jnp.where(supp, 0, 1).astype(jnp.int32)

            nev = jnp.where(hit, _NEG, ev)
            fsc[pl.ds(eoff, 16)] = nev
            nbv = jnp.where(it == (bnum & 15), jnp.max(nev), bv)
            b0[pl.ds(pl.multiple_of(s * 16, 16), 16)] = nbv
            nsv = jnp.where(it == (s - sreg), jnp.max(nbv), v)
            b1[pl.ds(soff, 16)] = nsv
            rn = jnp.max(nsv)
            r0 = jnp.where(sreg == 0, rn, r0)
            r1 = jnp.where(sreg == 16, rn, r1)
            r2 = jnp.where(sreg == 32, rn, r2)
            r3 = jnp.where(sreg == 48, rn, r3)
            r4 = jnp.where(sreg == 64, rn, r4)
            m = jnp.maximum(jnp.maximum(jnp.maximum(r0, r1),
                                        jnp.maximum(r2, r3)), r4)
            return k, m, r0, r1, r2, r3, r4

        m0 = jnp.maximum(jnp.maximum(jnp.maximum(r0, r1),
                                     jnp.maximum(r2, r3)), r4)
        fin = lax.while_loop(cond, body,
                             (jnp.int32(0), m0, r0, r1, r2, r3, r4))
        kf = fin[0]

        c0 = _lane(fx0[pl.ds(0, 16)], 0, _NEG)
        c1 = _lane(fy0[pl.ds(0, 16)], 0, _NEG)
        c2 = _lane(fx1[pl.ds(0, 16)], 0, _NEG)
        c3 = _lane(fy1[pl.ds(0, 16)], 0, _NEG)
        c4 = _lane(fillb[...], 0, _NEG)

        for j in range(_SELP // 16):
            sl = pl.ds(j * 16, 16)
            live = (j * 16 + it) < kf
            rowbase = (j * 16 + it) * 5
            plsc.store_scatter(b2, [rowbase], jnp.where(live, sx0[sl], c0))
            plsc.store_scatter(b2, [rowbase + 1], jnp.where(live, sy0[sl], c1))
            plsc.store_scatter(b2, [rowbase + 2], jnp.where(live, sx1[sl], c2))
            plsc.store_scatter(b2, [rowbase + 3], jnp.where(live, sy1[sl], c3))
            plsc.store_scatter(b2, [rowbase + 4], jnp.where(live, ssc[sl], c4))
        pltpu.sync_copy(b2.at[pl.ds(0, _OUTP)], out_hbm)


@jax.jit
def kernel(cls_logits, bbox_pred, anchors, image_h, image_w):
    pad = ((0, 0), (0, _NP - _N))
    allp = jnp.concatenate([
        jnp.pad(cls_logits[0].T, pad),
        jnp.pad(bbox_pred[0].T, pad),
        jnp.pad(anchors.T, pad),
    ], axis=0).reshape(-1)
    dims = jnp.zeros((16,), jnp.float32)
    dims = dims.at[0].set(jnp.float32(image_w)).at[1].set(jnp.float32(image_h))

    mesh = plsc.VectorSubcoreMesh(core_axis_name="c", subcore_axis_name="s",
                                  num_cores=1)
    run = pl.kernel(
        _sc_body,
        out_type=jax.ShapeDtypeStruct((_OUTP,), jnp.float32),
        mesh=mesh,
        compiler_params=pltpu.CompilerParams(needs_layout_passes=False),
        scratch_types=(
            [pltpu.VMEM((_CHUNK,), jnp.float32) for _ in range(10)] +
            [pltpu.VMEM((16,), jnp.float32), pltpu.VMEM((16,), jnp.float32)] +
            [pltpu.VMEM((_NP,), jnp.float32) for _ in range(5)] +
            [pltpu.VMEM((_SELP,), jnp.float32) for _ in range(5)] +
            [pltpu.VMEM_SHARED((_NP,), jnp.float32) for _ in range(5)] +
            [pltpu.VMEM_SHARED((_NBLK,), jnp.float32),
             pltpu.SemaphoreType.DMA, pltpu.SemaphoreType.DMA]
        ),
    )
    out = run(allp, dims)
    return out[:_MAXOUT * 5].reshape(_MAXOUT, 5)


# speedup vs baseline: 2.2188x; 1.0695x over previous
import jax
import jax.numpy as jnp
from jax import lax
from jax.experimental import pallas as pl
from jax.experimental.pallas import tpu as pltpu
from jax.experimental.pallas import tpu_sc as plsc

_CV = 0.1
_SV = 0.2
_CONF = 0.01
_NMS_T = 0.5
_MAXOUT = 100
_N = 20000
_NP = 20480
_NW = 16
_CHUNK = _NP // _NW
_VPC = _CHUNK // 16
_NBLK = _NP // 16
_NSUP = _NBLK // 16
_SELP = 112
_OUTP = 560
_NEG = -1e9
_BIG = 1 << 30


def _iota16():
    return lax.broadcasted_iota(jnp.int32, (16,), 0)


def _lane(v, lane_idx, fill):
    return jnp.max(jnp.where(_iota16() == lane_idx, v, fill))


def _sc_body(ph, dimh,
             out_hbm,
             b0, b1, b2, b3, b4, b5, b6, b7, b8, b9, dimv, fillb,
             fsc, fx0, fy0, fx1, fy1,
             sx0, sy0, sx1, sy1, ssc,
             sh0, sh1, sh2, sh3, sh4, shb, sem, semb):
    w = lax.axis_index("s")
    base = w * _CHUNK
    it = _iota16()

    cops = tuple(
        pltpu.async_copy(ph.at[pl.ds(f * _NP + base, _CHUNK)], ref, sem)
        for f, ref in enumerate((b0, b1, b2, b3, b4, b5, b6, b7, b8, b9))
    ) + (pltpu.async_copy(dimh, dimv, sem),)
    for c in cops:
        c.wait()
    dv = dimv[...]
    sw = jnp.max(jnp.where(it == 0, dv, _NEG))
    sh = jnp.max(jnp.where(it == 1, dv, _NEG))

    a0 = b0[pl.ds(0, 16)]
    a1 = b1[pl.ds(0, 16)]
    mx0 = jnp.maximum(a0, a1)
    fillb[...] = jnp.exp(a1 - mx0) / (jnp.exp(a0 - mx0) + jnp.exp(a1 - mx0))

    def decode(i, carry):
        off = pl.multiple_of(i * 16, 16)
        sl = pl.ds(off, 16)
        a = b0[sl]
        b = b1[sl]
        tcx = b2[sl]
        tcy = b3[sl]
        tw = b4[sl]
        th = b5[sl]
        acx = b6[sl]
        acy = b7[sl]
        aw = b8[sl]
        ah = b9[sl]
        mx = jnp.maximum(a, b)
        e0 = jnp.exp(a - mx)
        e1 = jnp.exp(b - mx)
        fg = e1 / (e0 + e1)
        cx = tcx * _CV * aw + acx
        cy = tcy * _CV * ah + acy
        bw = jnp.exp(tw * _SV) * aw
        bh = jnp.exp(th * _SV) * ah
        gidx = base + off + it
        masked = jnp.where((fg > _CONF) & (gidx < _N), fg, _NEG)
        b0[sl] = masked
        b1[sl] = (cx - bw / 2.0) * sw
        b2[sl] = (cy - bh / 2.0) * sh
        b3[sl] = (cx + bw / 2.0) * sw
        b4[sl] = (cy + bh / 2.0) * sh
        return carry

    lax.fori_loop(0, _VPC, decode, 0)

    for s in range(_VPC // 16):
        acc = plsc.load_gather(b0, [s * 256 + it * 16])
        for j in range(1, 16):
            acc = jnp.maximum(acc,
                              plsc.load_gather(b0, [s * 256 + it * 16 + j]))
        b5[pl.ds(s * 16, 16)] = acc

    sops = (
        pltpu.async_copy(b0, sh0.at[pl.ds(base, _CHUNK)], sem),
        pltpu.async_copy(b1, sh1.at[pl.ds(base, _CHUNK)], sem),
        pltpu.async_copy(b2, sh2.at[pl.ds(base, _CHUNK)], sem),
        pltpu.async_copy(b3, sh3.at[pl.ds(base, _CHUNK)], sem),
        pltpu.async_copy(b4, sh4.at[pl.ds(base, _CHUNK)], sem),
        pltpu.async_copy(b5.at[pl.ds(0, _VPC)],
                         shb.at[pl.ds(w * _VPC, _VPC)], sem),
    )
    for c in sops:
        c.wait()
    plsc.subcore_barrier()

    @pl.when(w == 0)
    def _phase_b():
        bop = pltpu.async_copy(shb, b0, semb)
        gops = (
            pltpu.async_copy(sh0, fsc, sem),
            pltpu.async_copy(sh1, fx0, sem),
            pltpu.async_copy(sh2, fy0, sem),
            pltpu.async_copy(sh3, fx1, sem),
            pltpu.async_copy(sh4, fy1, sem),
        )
        bop.wait()

        for si in range(_NSUP // 16):
            gbase = si * 256
            acc = plsc.load_gather(b0, [gbase + it * 16])
            for j in range(1, 16):
                acc = jnp.maximum(
                    acc, plsc.load_gather(b0, [gbase + it * 16 + j]))
            b1[pl.ds(si * 16, 16)] = acc

        r0 = jnp.max(b1[pl.ds(0, 16)])
        r1 = jnp.max(b1[pl.ds(16, 16)])
        r2 = jnp.max(b1[pl.ds(32, 16)])
        r3 = jnp.max(b1[pl.ds(48, 16)])
        r4 = jnp.max(b1[pl.ds(64, 16)])

        for c in gops:
            c.wait()

        empty = jnp.broadcast_to(jnp.float32(-1e6), (16,))
        for j in range(_SELP // 16):
            sl = pl.ds(j * 16, 16)
            sx0[sl] = empty
            sy0[sl] = empty
            sx1[sl] = empty
            sy1[sl] = empty
            ssc[sl] = empty

        def cond(state):
            k, m, _, _, _, _, _ = state
            return jnp.logical_and(k < _MAXOUT, m > _NEG)

        def body(state):
            k, m, r0, r1, r2, r3, r4 = state
            sreg = jnp.where(
                r0 == m, 0, jnp.where(
                    r1 == m, 16, jnp.where(
                        r2 == m, 32, jnp.where(r3 == m, 48, 64))))
            soff = pl.multiple_of(sreg, 16)
            v = b1[pl.ds(soff, 16)]
            s = sreg + plsc.all_reduce_ffs(v == m)[0]
            bv = b0[pl.ds(pl.multiple_of(s * 16, 16), 16)]
            bnum = s * 16 + plsc.all_reduce_ffs(bv == m)[0]
            eoff = pl.multiple_of(bnum * 16, 16)
            ev = fsc[pl.ds(eoff, 16)]
            lane_g = plsc.all_reduce_ffs(ev == m)

            hit = it == lane_g
            gidx = eoff + lane_g
            cx0 = plsc.load_gather(fx0, [gidx])
            cy0 = plsc.load_gather(fy0, [gidx])
            cx1 = plsc.load_gather(fx1, [gidx])
            cy1 = plsc.load_gather(fy1, [gidx])
            a1 = (jnp.maximum(cx1 - cx0, 0.0) *
                  jnp.maximum(cy1 - cy0, 0.0))

            def iou_step(j, acc):
                off = pl.multiple_of(j * 16, 16)
                tx0 = sx0[pl.ds(off, 16)]
                ty0 = sy0[pl.ds(off, 16)]
                tx1 = sx1[pl.ds(off, 16)]
                ty1 = sy1[pl.ds(off, 16)]
                iw = jnp.maximum(
                    jnp.minimum(cx1, tx1) - jnp.maximum(cx0, tx0), 0.0)
                ih = jnp.maximum(
                    jnp.minimum(cy1, ty1) - jnp.maximum(cy0, ty0), 0.0)
                inter = iw * ih
                a2 = (jnp.maximum(tx1 - tx0, 0.0) *
                      jnp.maximum(ty1 - ty0, 0.0))
                iou = inter / (a1 + a2 - inter + 1e-9)
                return jnp.maximum(acc, iou)

            nv = (k + 15) // 16
            iou_max = lax.fori_loop(0, nv, iou_step,
                                    jnp.zeros((16,), jnp.float32))
            supp = plsc.all_reduce_population_count(iou_max > _NMS_T)[0] > 0

            @pl.when(jnp.logical_not(supp))
            def _insert():
                koff = pl.multiple_of((k // 16) * 16, 16)
                klane = k & 15
                put = it == klane
                ksl = pl.ds(koff, 16)
                sx0[ksl] = jnp.where(put, cx0, sx0[ksl])
                sy0[ksl] = jnp.where(put, cy0, sy0[ksl])
                sx1[ksl] = jnp.where(put, cx1, sx1[ksl])
                sy1[ksl] = jnp.where(put, cy1, sy1[ksl])
                ssc[ksl] = jnp.where(put, m, ssc[ksl])

            k = k + jnp.where(supp, 0, 1).astype(jnp.int32)

            nev = jnp.where(hit, _NEG, ev)
            fsc[pl.ds(eoff, 16)] = nev
            nbv = jnp.where(it == (bnum & 15), jnp.max(nev), bv)
            b0[pl.ds(pl.multiple_of(s * 16, 16), 16)] = nbv
            nsv = jnp.where(it == (s - sreg), jnp.max(nbv), v)
            b1[pl.ds(soff, 16)] = nsv
            rn = jnp.max(nsv)
            r0 = jnp.where(sreg == 0, rn, r0)
            r1 = jnp.where(sreg == 16, rn, r1)
            r2 = jnp.where(sreg == 32, rn, r2)
            r3 = jnp.where(sreg == 48, rn, r3)
            r4 = jnp.where(sreg == 64, rn, r4)
            m = jnp.maximum(jnp.maximum(jnp.maximum(r0, r1),
                                        jnp.maximum(r2, r3)), r4)
            return k, m, r0, r1, r2, r3, r4

        m0 = jnp.maximum(jnp.maximum(jnp.maximum(r0, r1),
                                     jnp.maximum(r2, r3)), r4)
        fin = lax.while_loop(cond, body,
                             (jnp.int32(0), m0, r0, r1, r2, r3, r4))
        kf = fin[0]

        c0 = _lane(fx0[pl.ds(0, 16)], 0, _NEG)
        c1 = _lane(fy0[pl.ds(0, 16)], 0, _NEG)
        c2 = _lane(fx1[pl.ds(0, 16)], 0, _NEG)
        c3 = _lane(fy1[pl.ds(0, 16)], 0, _NEG)
        c4 = _lane(fillb[...], 0, _NEG)

        for j in range(_SELP // 16):
            sl = pl.ds(j * 16, 16)
            live = (j * 16 + it) < kf
            rowbase = (j * 16 + it) * 5
            plsc.store_scatter(b2, [rowbase], jnp.where(live, sx0[sl], c0))
            plsc.store_scatter(b2, [rowbase + 1], jnp.where(live, sy0[sl], c1))
            plsc.store_scatter(b2, [rowbase + 2], jnp.where(live, sx1[sl], c2))
            plsc.store_scatter(b2, [rowbase + 3], jnp.where(live, sy1[sl], c3))
            plsc.store_scatter(b2, [rowbase + 4], jnp.where(live, ssc[sl], c4))
        pltpu.sync_copy(b2.at[pl.ds(0, _OUTP)], out_hbm)


@jax.jit
def kernel(cls_logits, bbox_pred, anchors, image_h, image_w):
    pad = ((0, 0), (0, _NP - _N))
    allp = jnp.concatenate([
        jnp.pad(cls_logits[0].T, pad),
        jnp.pad(bbox_pred[0].T, pad),
        jnp.pad(anchors.T, pad),
    ], axis=0).reshape(-1)
    dims = jnp.zeros((16,), jnp.float32)
    dims = dims.at[0].set(jnp.float32(image_w)).at[1].set(jnp.float32(image_h))

    mesh = plsc.VectorSubcoreMesh(core_axis_name="c", subcore_axis_name="s",
                                  num_cores=1)
    run = pl.kernel(
        _sc_body,
        out_type=jax.ShapeDtypeStruct((_OUTP,), jnp.float32),
        mesh=mesh,
        compiler_params=pltpu.CompilerParams(needs_layout_passes=False),
        scratch_types=(
            [pltpu.VMEM((_CHUNK,), jnp.float32) for _ in range(10)] +
            [pltpu.VMEM((16,), jnp.float32), pltpu.VMEM((16,), jnp.float32)] +
            [pltpu.VMEM((_NP,), jnp.float32) for _ in range(5)] +
            [pltpu.VMEM((_SELP,), jnp.float32) for _ in range(5)] +
            [pltpu.VMEM_SHARED((_NP,), jnp.float32) for _ in range(5)] +
            [pltpu.VMEM_SHARED((_NBLK,), jnp.float32),
             pltpu.SemaphoreType.DMA, pltpu.SemaphoreType.DMA]
        ),
    )
    out = run(allp, dims)
    return out[:_MAXOUT * 5].reshape(_MAXOUT, 5)


# speedup vs baseline: 2.2978x; 1.0356x over previous
import jax
import jax.numpy as jnp
from jax import lax
from jax.experimental import pallas as pl
from jax.experimental.pallas import tpu as pltpu
from jax.experimental.pallas import tpu_sc as plsc

_CV = 0.1
_SV = 0.2
_CONF = 0.01
_NMS_T = 0.5
_MAXOUT = 100
_N = 20000
_NP = 20480
_NW = 16
_CHUNK = _NP // _NW
_VPC = _CHUNK // 16
_NBLK = _NP // 16
_NSUP = _NBLK // 16
_SELP = 112
_OUTP = 560
_NEG = -1e9
_BIG = 1 << 30


def _iota16():
    return lax.broadcasted_iota(jnp.int32, (16,), 0)


def _lane(v, lane_idx, fill):
    return jnp.max(jnp.where(_iota16() == lane_idx, v, fill))


def _sc_body(ph, dimh,
             out_hbm,
             b0, b1, b2, b3, b4, b5, b6, b7, b8, b9, dimv, fillb,
             fsc, fx0, fy0, fx1, fy1,
             sx0, sy0, sx1, sy1, ssc,
             sh0, sh1, sh2, sh3, sh4, shb, sem, semb):
    w = lax.axis_index("s")
    base = w * _CHUNK
    it = _iota16()

    cops = tuple(
        pltpu.async_copy(ph.at[pl.ds(f * _NP + base, _CHUNK)], ref, sem)
        for f, ref in enumerate((b0, b1, b2, b3, b4, b5, b6, b7, b8, b9))
    ) + (pltpu.async_copy(dimh, dimv, sem),)
    for c in cops:
        c.wait()
    dv = dimv[...]
    sw = jnp.max(jnp.where(it == 0, dv, _NEG))
    sh = jnp.max(jnp.where(it == 1, dv, _NEG))

    a0 = b0[pl.ds(0, 16)]
    a1 = b1[pl.ds(0, 16)]
    mx0 = jnp.maximum(a0, a1)
    fillb[...] = jnp.exp(a1 - mx0) / (jnp.exp(a0 - mx0) + jnp.exp(a1 - mx0))

    def decode(i, carry):
        off = pl.multiple_of(i * 16, 16)
        sl = pl.ds(off, 16)
        a = b0[sl]
        b = b1[sl]
        tcx = b2[sl]
        tcy = b3[sl]
        tw = b4[sl]
        th = b5[sl]
        acx = b6[sl]
        acy = b7[sl]
        aw = b8[sl]
        ah = b9[sl]
        mx = jnp.maximum(a, b)
        e0 = jnp.exp(a - mx)
        e1 = jnp.exp(b - mx)
        fg = e1 / (e0 + e1)
        cx = tcx * _CV * aw + acx
        cy = tcy * _CV * ah + acy
        bw = jnp.exp(tw * _SV) * aw
        bh = jnp.exp(th * _SV) * ah
        gidx = base + off + it
        masked = jnp.where((fg > _CONF) & (gidx < _N), fg, _NEG)
        b0[sl] = masked
        b1[sl] = (cx - bw / 2.0) * sw
        b2[sl] = (cy - bh / 2.0) * sh
        b3[sl] = (cx + bw / 2.0) * sw
        b4[sl] = (cy + bh / 2.0) * sh
        return carry

    lax.fori_loop(0, _VPC, decode, 0)

    for s in range(_VPC // 16):
        acc = plsc.load_gather(b0, [s * 256 + it * 16])
        for j in range(1, 16):
            acc = jnp.maximum(acc,
                              plsc.load_gather(b0, [s * 256 + it * 16 + j]))
        b5[pl.ds(s * 16, 16)] = acc

    sops = (
        pltpu.async_copy(b0, sh0.at[pl.ds(base, _CHUNK)], sem),
        pltpu.async_copy(b1, sh1.at[pl.ds(base, _CHUNK)], sem),
        pltpu.async_copy(b2, sh2.at[pl.ds(base, _CHUNK)], sem),
        pltpu.async_copy(b3, sh3.at[pl.ds(base, _CHUNK)], sem),
        pltpu.async_copy(b4, sh4.at[pl.ds(base, _CHUNK)], sem),
        pltpu.async_copy(b5.at[pl.ds(0, _VPC)],
                         shb.at[pl.ds(w * _VPC, _VPC)], sem),
    )
    for c in sops:
        c.wait()
    plsc.subcore_barrier()

    @pl.when(w == 0)
    def _phase_b():
        bop = pltpu.async_copy(shb, b0, semb)
        gops = (
            pltpu.async_copy(sh0, fsc, sem),
            pltpu.async_copy(sh1, fx0, sem),
            pltpu.async_copy(sh2, fy0, sem),
            pltpu.async_copy(sh3, fx1, sem),
            pltpu.async_copy(sh4, fy1, sem),
        )
        bop.wait()

        for si in range(_NSUP // 16):
            gbase = si * 256
            acc = plsc.load_gather(b0, [gbase + it * 16])
            for j in range(1, 16):
                acc = jnp.maximum(
                    acc, plsc.load_gather(b0, [gbase + it * 16 + j]))
            b1[pl.ds(si * 16, 16)] = acc

        r0 = jnp.max(b1[pl.ds(0, 16)])
        r1 = jnp.max(b1[pl.ds(16, 16)])
        r2 = jnp.max(b1[pl.ds(32, 16)])
        r3 = jnp.max(b1[pl.ds(48, 16)])
        r4 = jnp.max(b1[pl.ds(64, 16)])

        for c in gops:
            c.wait()

        empty = jnp.broadcast_to(jnp.float32(-1e6), (16,))
        for j in range(_SELP // 16):
            sl = pl.ds(j * 16, 16)
            sx0[sl] = empty
            sy0[sl] = empty
            sx1[sl] = empty
            sy1[sl] = empty
            ssc[sl] = empty

        def cond(state):
            k, m, _, _, _, _, _ = state
            return jnp.logical_and(k < _MAXOUT, m > _NEG)

        def body(state):
            k, m, r0, r1, r2, r3, r4 = state
            sreg = jnp.where(
                r0 == m, 0, jnp.where(
                    r1 == m, 16, jnp.where(
                        r2 == m, 32, jnp.where(r3 == m, 48, 64))))
            soff = pl.multiple_of(sreg, 16)
            v = b1[pl.ds(soff, 16)]
            s = sreg + plsc.all_reduce_ffs(v == m)[0]
            bv = b0[pl.ds(pl.multiple_of(s * 16, 16), 16)]
            bnum = s * 16 + plsc.all_reduce_ffs(bv == m)[0]
            eoff = pl.multiple_of(bnum * 16, 16)
            ev = fsc[pl.ds(eoff, 16)]
            lane_g = plsc.all_reduce_ffs(ev == m)

            hit = it == lane_g
            gidx = eoff + lane_g
            cx0 = plsc.load_gather(fx0, [gidx])
            cy0 = plsc.load_gather(fy0, [gidx])
            cx1 = plsc.load_gather(fx1, [gidx])
            cy1 = plsc.load_gather(fy1, [gidx])
            a1 = (jnp.maximum(cx1 - cx0, 0.0) *
                  jnp.maximum(cy1 - cy0, 0.0))

            def iou_step(j, acc):
                off = pl.multiple_of(j * 16, 16)
                tx0 = sx0[pl.ds(off, 16)]
                ty0 = sy0[pl.ds(off, 16)]
                tx1 = sx1[pl.ds(off, 16)]
                ty1 = sy1[pl.ds(off, 16)]
                iw = jnp.maximum(
                    jnp.minimum(cx1, tx1) - jnp.maximum(cx0, tx0), 0.0)
                ih = jnp.maximum(
                    jnp.minimum(cy1, ty1) - jnp.maximum(cy0, ty0), 0.0)
                inter = iw * ih
                a2 = (jnp.maximum(tx1 - tx0, 0.0) *
                      jnp.maximum(ty1 - ty0, 0.0))
                iou = inter / (a1 + a2 - inter + 1e-9)
                return jnp.maximum(acc, iou)

            iou_max = jnp.zeros((16,), jnp.float32)
            for j in range(_SELP // 16):
                iou_max = iou_step(j, iou_max)
            supp = plsc.all_reduce_population_count(iou_max > _NMS_T)[0] > 0

            @pl.when(jnp.logical_not(supp))
            def _insert():
                koff = pl.multiple_of((k // 16) * 16, 16)
                klane = k & 15
                put = it == klane
                ksl = pl.ds(koff, 16)
                sx0[ksl] = jnp.where(put, cx0, sx0[ksl])
                sy0[ksl] = jnp.where(put, cy0, sy0[ksl])
                sx1[ksl] = jnp.where(put, cx1, sx1[ksl])
                sy1[ksl] = jnp.where(put, cy1, sy1[ksl])
                ssc[ksl] = jnp.where(put, m, ssc[ksl])

            k = k + jnp.where(supp, 0, 1).astype(jnp.int32)

            nev = jnp.where(hit, _NEG, ev)
            fsc[pl.ds(eoff, 16)] = nev
            nbv = jnp.where(it == (bnum & 15), jnp.max(nev), bv)
            b0[pl.ds(pl.multiple_of(s * 16, 16), 16)] = nbv
            nsv = jnp.where(it == (s - sreg), jnp.max(nbv), v)
            b1[pl.ds(soff, 16)] = nsv
            rn = jnp.max(nsv)
            r0 = jnp.where(sreg == 0, rn, r0)
            r1 = jnp.where(sreg == 16, rn, r1)
            r2 = jnp.where(sreg == 32, rn, r2)
            r3 = jnp.where(sreg == 48, rn, r3)
            r4 = jnp.where(sreg == 64, rn, r4)
            m = jnp.maximum(jnp.maximum(jnp.maximum(r0, r1),
                                        jnp.maximum(r2, r3)), r4)
            return k, m, r0, r1, r2, r3, r4

        m0 = jnp.maximum(jnp.maximum(jnp.maximum(r0, r1),
                                     jnp.maximum(r2, r3)), r4)
        fin = lax.while_loop(cond, body,
                             (jnp.int32(0), m0, r0, r1, r2, r3, r4))
        kf = fin[0]

        c0 = _lane(fx0[pl.ds(0, 16)], 0, _NEG)
        c1 = _lane(fy0[pl.ds(0, 16)], 0, _NEG)
        c2 = _lane(fx1[pl.ds(0, 16)], 0, _NEG)
        c3 = _lane(fy1[pl.ds(0, 16)], 0, _NEG)
        c4 = _lane(fillb[...], 0, _NEG)

        for j in range(_SELP // 16):
            sl = pl.ds(j * 16, 16)
            live = (j * 16 + it) < kf
            rowbase = (j * 16 + it) * 5
            plsc.store_scatter(b2, [rowbase], jnp.where(live, sx0[sl], c0))
            plsc.store_scatter(b2, [rowbase + 1], jnp.where(live, sy0[sl], c1))
            plsc.store_scatter(b2, [rowbase + 2], jnp.where(live, sx1[sl], c2))
            plsc.store_scatter(b2, [rowbase + 3], jnp.where(live, sy1[sl], c3))
            plsc.store_scatter(b2, [rowbase + 4], jnp.where(live, ssc[sl], c4))
        pltpu.sync_copy(b2.at[pl.ds(0, _OUTP)], out_hbm)


@jax.jit
def kernel(cls_logits, bbox_pred, anchors, image_h, image_w):
    pad = ((0, 0), (0, _NP - _N))
    allp = jnp.concatenate([
        jnp.pad(cls_logits[0].T, pad),
        jnp.pad(bbox_pred[0].T, pad),
        jnp.pad(anchors.T, pad),
    ], axis=0).reshape(-1)
    dims = jnp.zeros((16,), jnp.float32)
    dims = dims.at[0].set(jnp.float32(image_w)).at[1].set(jnp.float32(image_h))

    mesh = plsc.VectorSubcoreMesh(core_axis_name="c", subcore_axis_name="s",
                                  num_cores=1)
    run = pl.kernel(
        _sc_body,
        out_type=jax.ShapeDtypeStruct((_OUTP,), jnp.float32),
        mesh=mesh,
        compiler_params=pltpu.CompilerParams(needs_layout_passes=False),
        scratch_types=(
            [pltpu.VMEM((_CHUNK,), jnp.float32) for _ in range(10)] +
            [pltpu.VMEM((16,), jnp.float32), pltpu.VMEM((16,), jnp.float32)] +
            [pltpu.VMEM((_NP,), jnp.float32) for _ in range(5)] +
            [pltpu.VMEM((_SELP,), jnp.float32) for _ in range(5)] +
            [pltpu.VMEM_SHARED((_NP,), jnp.float32) for _ in range(5)] +
            [pltpu.VMEM_SHARED((_NBLK,), jnp.float32),
             pltpu.SemaphoreType.DMA, pltpu.SemaphoreType.DMA]
        ),
    )
    out = run(allp, dims)
    return out[:_MAXOUT * 5].reshape(_MAXOUT, 5)
